# baseline (speedup 1.0000x reference)
import functools

import jax
import jax.numpy as jnp
from jax import lax
from jax.experimental import pallas as pl
from jax.experimental.pallas import tpu as pltpu
from jax.experimental.pallas import tpu_sc as plsc

L = 16

_GATHER_DNUMS = lax.GatherDimensionNumbers(
    offset_dims=(), collapsed_slice_dims=(0,), start_index_map=(0,))


def _bcast_lane(vec, lane):
    idx = jnp.full((L, 1), lane, jnp.int32)
    return lax.gather(vec, idx, _GATHER_DNUMS, (1,),
                      mode=lax.GatherScatterMode.PROMISE_IN_BOUNDS)


def _butterfly_sum(r, lanes):
    for k in (1, 2, 4, 8):
        perm = jnp.bitwise_xor(lanes, k).reshape(L, 1)
        r = r + lax.gather(r, perm, _GATHER_DNUMS, (1,),
                           mode=lax.GatherScatterMode.PROMISE_IN_BOUNDS)
    return r


def _fold_weights(W1, W2, W3, b1, b2, b3, fm_bias):
    def body(w1_ref, w2_ref, w3_ref, b1_ref, b2_ref, b3_ref, fmb_ref,
             weff_ref, btot_ref):
        w23 = jnp.dot(w2_ref[...], w3_ref[...],
                      preferred_element_type=jnp.float32)
        weff_ref[...] = jnp.dot(w1_ref[...], w23,
                                preferred_element_type=jnp.float32)
        btot = (jnp.dot(b1_ref[...], w23, preferred_element_type=jnp.float32)
                + jnp.dot(b2_ref[...], w3_ref[...],
                          preferred_element_type=jnp.float32))
        btot_ref[...] = btot + b3_ref[...] + fmb_ref[...]

    fe = W1.shape[0]
    weff, btot = pl.pallas_call(
        body,
        out_shape=(
            jax.ShapeDtypeStruct((fe, 1), jnp.float32),
            jax.ShapeDtypeStruct((1, 1), jnp.float32),
        ),
    )(W1, W2, W3, b1.reshape(1, -1), b2.reshape(1, -1), b3.reshape(1, 1),
      jnp.reshape(fm_bias, (1, 1)).astype(jnp.float32))
    return weff, btot


def _repack_table(table_t, V, E, BV=2048):
    G = 128 // E

    def body(t_ref, out_ref):
        x = t_ref[...]
        ri = lax.broadcasted_iota(jnp.int32, (128 // G, 128), 0)
        ci = lax.broadcasted_iota(jnp.int32, (128 // G, 128), 1)
        sel = [(ci == G * ri + a).astype(jnp.float32) for a in range(G)]
        outs = []
        for t in range(BV // 128):
            m1 = x[:, 128 * t:128 * (t + 1)].T
            blk = jnp.concatenate(
                [jnp.dot(sel[a], m1, preferred_element_type=jnp.float32)
                 for a in range(G)], axis=1)
            outs.append(blk)
        out_ref[...] = jnp.concatenate(outs, axis=0)

    return pl.pallas_call(
        body,
        grid=(pl.cdiv(V, BV),),
        in_specs=[pl.BlockSpec((E, BV), lambda i: (0, i))],
        out_specs=pl.BlockSpec((BV * E // 128, 128), lambda i: (i, 0)),
        out_shape=jax.ShapeDtypeStruct((V * E // 128, 128), jnp.float32),
    )(table_t)


def _make_sc_kernel(B, F, E, NW):
    SPW = B // NW
    NG = SPW // L
    mesh = plsc.VectorSubcoreMesh(core_axis_name="c", subcore_axis_name="s")

    @functools.partial(
        pl.kernel,
        out_type=jax.ShapeDtypeStruct((B,), jnp.float32),
        mesh=mesh,
        scratch_types=[
            pltpu.VMEM((F, SPW), jnp.int32),
            pltpu.VMEM((F * SPW,), jnp.float32),
            pltpu.VMEM((F, SPW, E), jnp.float32),
            pltpu.VMEM((F, SPW), jnp.float32),
            pltpu.VMEM((F, E), jnp.float32),
            pltpu.VMEM((L,), jnp.float32),
            pltpu.VMEM((SPW,), jnp.float32),
            pltpu.SemaphoreType.DMA,
            pltpu.SemaphoreType.DMA,
        ],
        compiler_params=pltpu.CompilerParams(use_tc_tiling_on_sc=False),
    )
    def sc_kernel(idx_hbm, vals_hbm, emb_hbm, fmw_hbm, weff_hbm, btot_hbm,
                  out_hbm, idx_v, vals_v, rows_v, fw_v, weff_v, btot_v, out_v,
                  sem_rows, sem_fw):
        wid = lax.axis_index("s") * 2 + lax.axis_index("c")
        base = pl.multiple_of(wid * SPW, SPW)

        pltpu.sync_copy(idx_hbm.at[wid], idx_v)
        pltpu.sync_copy(vals_hbm.at[wid], vals_v)
        pltpu.sync_copy(weff_hbm, weff_v)
        pltpu.sync_copy(btot_hbm, btot_v)

        H = SPW // 2
        halves = []
        for h0 in (0, H):
            hh = []
            for f in range(F):
                hh.append(pltpu.async_copy(
                    emb_hbm.at[idx_v.at[f, pl.ds(h0, H)]],
                    rows_v.at[f, pl.ds(h0, H)], sem_rows))
                hh.append(pltpu.async_copy(
                    fmw_hbm.at[idx_v.at[f, pl.ds(h0, H)]],
                    fw_v.at[f, pl.ds(h0, H)], sem_fw))
            halves.append(hh)

        lanes = lax.iota(jnp.int32, L)
        zero = jnp.zeros((L,), jnp.float32)
        btot = btot_v[...]

        def group_body(g, _):
            s0 = pl.multiple_of(g * L, L)

            def sample_body(l, outz):
                s = s0 + l
                lidx = jnp.full((L, 1), l, jnp.int32)
                a0 = a1 = q0 = q1 = d0 = d1 = zero
                for f in range(F):
                    e0 = rows_v[f, s, pl.ds(0, L)]
                    e1 = rows_v[f, s, pl.ds(L, L)]
                    vchunk = vals_v[pl.ds(f * SPW + s0, L)]
                    vb = lax.gather(
                        vchunk, lidx, _GATHER_DNUMS, (1,),
                        mode=lax.GatherScatterMode.PROMISE_IN_BOUNDS)
                    se0 = e0 * vb
                    se1 = e1 * vb
                    a0 = a0 + se0
                    a1 = a1 + se1
                    q0 = q0 + se0 * se0
                    q1 = q1 + se1 * se1
                    d0 = d0 + se0 * weff_v[f, pl.ds(0, L)]
                    d1 = d1 + se1 * weff_v[f, pl.ds(L, L)]
                r = d0 + d1 + 0.5 * (a0 * a0 + a1 * a1 - q0 - q1)
                r = _butterfly_sum(r, lanes)
                return jnp.where(lanes == l, r, outz)

            outz = lax.fori_loop(0, L, sample_body, zero)

            fm1 = zero
            for f in range(F):
                fm1 = fm1 + fw_v[f, pl.ds(s0, L)] * vals_v[pl.ds(f * SPW + s0, L)]

            zv = outz + fm1 + btot
            out_v[pl.ds(s0, L)] = 1.0 / (1.0 + jnp.exp(-zv))
            return 0

        for h in halves[0]:
            h.wait()
        lax.fori_loop(0, NG // 2, group_body, 0)
        for h in halves[1]:
            h.wait()
        lax.fori_loop(NG // 2, NG, group_body, 0)
        pltpu.sync_copy(out_v, out_hbm.at[pl.ds(base, SPW)])

    return sc_kernel


def kernel(idxs, vals, shared_emb_table, fm_w_table, fm_bias,
           W1, b1, W2, b2, W3, b3):
    B, F = idxs.shape
    V, E = shared_emb_table.shape
    NW = 32
    SPW = B // NW

    weff, btot = _fold_weights(W1, W2, W3, b1, b2, b3, fm_bias)
    tab = shared_emb_table

    idx_w = idxs.reshape(NW, SPW, F).transpose(0, 2, 1)
    vals_w = vals.reshape(NW, SPW, F).transpose(0, 2, 1).reshape(NW, F * SPW)

    sc = _make_sc_kernel(B, F, E, NW)
    out_flat = sc(idx_w, vals_w, tab, fm_w_table.reshape(-1),
                  weff.reshape(F, E), jnp.broadcast_to(btot.reshape(1), (L,)))
    return out_flat.reshape(B, 1)


# speedup vs baseline: 1.0254x; 1.0254x over previous
import functools

import jax
import jax.numpy as jnp
from jax import lax
from jax.experimental import pallas as pl
from jax.experimental.pallas import tpu as pltpu
from jax.experimental.pallas import tpu_sc as plsc

L = 16

_GATHER_DNUMS = lax.GatherDimensionNumbers(
    offset_dims=(), collapsed_slice_dims=(0,), start_index_map=(0,))


def _butterfly_sum(r, lanes):
    for k in (1, 2, 4, 8):
        perm = jnp.bitwise_xor(lanes, k).reshape(L, 1)
        r = r + lax.gather(r, perm, _GATHER_DNUMS, (1,),
                           mode=lax.GatherScatterMode.PROMISE_IN_BOUNDS)
    return r


def _fold_weights(W1, W2, W3, b1, b2, b3, fm_bias):
    def body(w1_ref, w2_ref, w3_ref, b1_ref, b2_ref, b3_ref, fmb_ref,
             weff_ref, btot_ref):
        w23 = jnp.dot(w2_ref[...], w3_ref[...],
                      preferred_element_type=jnp.float32)
        weff_ref[...] = jnp.dot(w1_ref[...], w23,
                                preferred_element_type=jnp.float32)
        btot = (jnp.dot(b1_ref[...], w23, preferred_element_type=jnp.float32)
                + jnp.dot(b2_ref[...], w3_ref[...],
                          preferred_element_type=jnp.float32))
        btot_ref[...] = btot + b3_ref[...] + fmb_ref[...]

    fe = W1.shape[0]
    weff, btot = pl.pallas_call(
        body,
        out_shape=(
            jax.ShapeDtypeStruct((fe, 1), jnp.float32),
            jax.ShapeDtypeStruct((1, 1), jnp.float32),
        ),
    )(W1, W2, W3, b1.reshape(1, -1), b2.reshape(1, -1), b3.reshape(1, 1),
      jnp.reshape(fm_bias, (1, 1)).astype(jnp.float32))
    return weff, btot


def _make_sc_kernel(B, F, E, NW):
    SPW = B // NW
    NG = SPW // L
    mesh = plsc.VectorSubcoreMesh(core_axis_name="c", subcore_axis_name="s")

    @functools.partial(
        pl.kernel,
        out_type=jax.ShapeDtypeStruct((B,), jnp.float32),
        mesh=mesh,
        scratch_types=[
            pltpu.VMEM((F, SPW), jnp.int32),
            pltpu.VMEM((F * SPW,), jnp.float32),
            pltpu.VMEM((F, SPW, E), jnp.float32),
            pltpu.VMEM((F, SPW), jnp.float32),
            pltpu.VMEM((F, E), jnp.float32),
            pltpu.VMEM((L,), jnp.float32),
            pltpu.VMEM((SPW,), jnp.float32),
            pltpu.SemaphoreType.DMA,
            pltpu.SemaphoreType.DMA,
        ],
        compiler_params=pltpu.CompilerParams(use_tc_tiling_on_sc=False),
    )
    def sc_kernel(idx_hbm, vals_hbm, emb_hbm, fmw_hbm, weff_hbm, btot_hbm,
                  out_hbm, idx_v, vals_v, rows_v, fw_v, weff_v, btot_v, out_v,
                  sem_rows, sem_fw):
        wid = lax.axis_index("s") * 2 + lax.axis_index("c")
        base = pl.multiple_of(wid * SPW, SPW)

        pltpu.sync_copy(idx_hbm.at[wid], idx_v)
        pltpu.sync_copy(vals_hbm.at[wid], vals_v)
        pltpu.sync_copy(weff_hbm, weff_v)
        pltpu.sync_copy(btot_hbm, btot_v)

        handles = []
        for f in range(F):
            handles.append(
                pltpu.async_copy(emb_hbm.at[idx_v.at[f]], rows_v.at[f],
                                 sem_rows))
            handles.append(
                pltpu.async_copy(fmw_hbm.at[idx_v.at[f]], fw_v.at[f], sem_fw))
        for h in handles:
            h.wait()

        lanes = lax.iota(jnp.int32, L)
        zero = jnp.zeros((L,), jnp.float32)
        btot = btot_v[...]

        def group_body(g, _):
            s0 = pl.multiple_of(g * L, L)

            def sample_body(l, outz):
                s = s0 + l
                lidx = jnp.full((L, 1), l, jnp.int32)
                a0 = a1 = q0 = q1 = d0 = d1 = zero
                for f in range(F):
                    e0 = rows_v[f, s, pl.ds(0, L)]
                    e1 = rows_v[f, s, pl.ds(L, L)]
                    vchunk = vals_v[pl.ds(f * SPW + s0, L)]
                    vb = lax.gather(
                        vchunk, lidx, _GATHER_DNUMS, (1,),
                        mode=lax.GatherScatterMode.PROMISE_IN_BOUNDS)
                    se0 = e0 * vb
                    se1 = e1 * vb
                    a0 = a0 + se0
                    a1 = a1 + se1
                    q0 = q0 + se0 * se0
                    q1 = q1 + se1 * se1
                    d0 = d0 + se0 * weff_v[f, pl.ds(0, L)]
                    d1 = d1 + se1 * weff_v[f, pl.ds(L, L)]
                r = d0 + d1 + 0.5 * (a0 * a0 + a1 * a1 - q0 - q1)
                r = _butterfly_sum(r, lanes)
                return jnp.where(lanes == l, r, outz)

            outz = lax.fori_loop(0, L, sample_body, zero)

            fm1 = zero
            for f in range(F):
                fm1 = fm1 + fw_v[f, pl.ds(s0, L)] * vals_v[pl.ds(f * SPW + s0, L)]

            zv = outz + fm1 + btot
            out_v[pl.ds(s0, L)] = 1.0 / (1.0 + jnp.exp(-zv))
            return 0

        lax.fori_loop(0, NG, group_body, 0)
        pltpu.sync_copy(out_v, out_hbm.at[pl.ds(base, SPW)])

    return sc_kernel


def kernel(idxs, vals, shared_emb_table, fm_w_table, fm_bias,
           W1, b1, W2, b2, W3, b3):
    B, F = idxs.shape
    V, E = shared_emb_table.shape
    NW = 32
    SPW = B // NW

    weff, btot = _fold_weights(W1, W2, W3, b1, b2, b3, fm_bias)

    idx_w = idxs.reshape(NW, SPW, F).transpose(0, 2, 1)
    vals_w = vals.reshape(NW, SPW, F).transpose(0, 2, 1).reshape(NW, F * SPW)

    sc = _make_sc_kernel(B, F, E, NW)
    out_flat = sc(idx_w, vals_w, shared_emb_table, fm_w_table.reshape(-1),
                  weff.reshape(F, E), jnp.broadcast_to(btot.reshape(1), (L,)))
    return out_flat.reshape(B, 1)


# speedup vs baseline: 1.0445x; 1.0187x over previous
import functools

import jax
import jax.numpy as jnp
from jax import lax
from jax.experimental import pallas as pl
from jax.experimental.pallas import tpu as pltpu
from jax.experimental.pallas import tpu_sc as plsc

L = 16

_GATHER_DNUMS = lax.GatherDimensionNumbers(
    offset_dims=(), collapsed_slice_dims=(0,), start_index_map=(0,))


def _butterfly_sum(r, lanes):
    for k in (1, 2, 4, 8):
        perm = jnp.bitwise_xor(lanes, k).reshape(L, 1)
        r = r + lax.gather(r, perm, _GATHER_DNUMS, (1,),
                           mode=lax.GatherScatterMode.PROMISE_IN_BOUNDS)
    return r


def _fold_weights(W1, W2, W3, b1, b2, b3, fm_bias):
    def body(w1_ref, w2_ref, w3_ref, b1_ref, b2_ref, b3_ref, fmb_ref,
             weff_ref, btot_ref):
        w23 = jnp.dot(w2_ref[...], w3_ref[...],
                      preferred_element_type=jnp.float32)
        weff_ref[...] = jnp.dot(w1_ref[...], w23,
                                preferred_element_type=jnp.float32)
        btot = (jnp.dot(b1_ref[...], w23, preferred_element_type=jnp.float32)
                + jnp.dot(b2_ref[...], w3_ref[...],
                          preferred_element_type=jnp.float32))
        btot_ref[...] = btot + b3_ref[...] + fmb_ref[...]

    fe = W1.shape[0]
    weff, btot = pl.pallas_call(
        body,
        out_shape=(
            jax.ShapeDtypeStruct((fe, 1), jnp.float32),
            jax.ShapeDtypeStruct((1, 1), jnp.float32),
        ),
    )(W1, W2, W3, b1.reshape(1, -1), b2.reshape(1, -1), b3.reshape(1, 1),
      jnp.reshape(fm_bias, (1, 1)).astype(jnp.float32))
    return weff, btot


def _make_sc_kernel(B, F, E, NW):
    SPW = B // NW
    NG = SPW // L
    mesh = plsc.VectorSubcoreMesh(core_axis_name="c", subcore_axis_name="s")

    @functools.partial(
        pl.kernel,
        out_type=jax.ShapeDtypeStruct((B,), jnp.float32),
        mesh=mesh,
        scratch_types=[
            pltpu.VMEM((F, SPW), jnp.int32),
            pltpu.VMEM((F * SPW,), jnp.float32),
            pltpu.VMEM((F, SPW, E), jnp.float32),
            pltpu.VMEM((F, SPW), jnp.float32),
            pltpu.VMEM((F, E), jnp.float32),
            pltpu.VMEM((L,), jnp.float32),
            pltpu.VMEM((SPW,), jnp.float32),
            pltpu.SemaphoreType.DMA,
            pltpu.SemaphoreType.DMA,
        ],
        compiler_params=pltpu.CompilerParams(use_tc_tiling_on_sc=False),
    )
    def sc_kernel(idx_hbm, vals_hbm, emb_hbm, fmw_hbm, weff_hbm, btot_hbm,
                  out_hbm, idx_v, vals_v, rows_v, fw_v, weff_v, btot_v, out_v,
                  sem_rows, sem_fw):
        wid = lax.axis_index("s") * 2 + lax.axis_index("c")
        base = pl.multiple_of(wid * SPW, SPW)

        pltpu.sync_copy(idx_hbm.at[wid], idx_v)

        handles = []
        for f in range(F):
            handles.append(
                pltpu.async_copy(emb_hbm.at[idx_v.at[f]], rows_v.at[f],
                                 sem_rows))
            handles.append(
                pltpu.async_copy(fmw_hbm.at[idx_v.at[f]], fw_v.at[f], sem_fw))
        pltpu.sync_copy(vals_hbm.at[wid], vals_v)
        pltpu.sync_copy(weff_hbm, weff_v)
        pltpu.sync_copy(btot_hbm, btot_v)
        for h in handles:
            h.wait()

        lanes = lax.iota(jnp.int32, L)
        zero = jnp.zeros((L,), jnp.float32)
        btot = btot_v[...]

        def group_body(g, _):
            s0 = pl.multiple_of(g * L, L)

            def sample_body(l, outz):
                s = s0 + l
                lidx = jnp.full((L, 1), l, jnp.int32)
                a0 = a1 = q0 = q1 = d0 = d1 = zero
                for f in range(F):
                    e0 = rows_v[f, s, pl.ds(0, L)]
                    e1 = rows_v[f, s, pl.ds(L, L)]
                    vchunk = vals_v[pl.ds(f * SPW + s0, L)]
                    vb = lax.gather(
                        vchunk, lidx, _GATHER_DNUMS, (1,),
                        mode=lax.GatherScatterMode.PROMISE_IN_BOUNDS)
                    se0 = e0 * vb
                    se1 = e1 * vb
                    a0 = a0 + se0
                    a1 = a1 + se1
                    q0 = q0 + se0 * se0
                    q1 = q1 + se1 * se1
                    d0 = d0 + se0 * weff_v[f, pl.ds(0, L)]
                    d1 = d1 + se1 * weff_v[f, pl.ds(L, L)]
                r = d0 + d1 + 0.5 * (a0 * a0 + a1 * a1 - q0 - q1)
                r = _butterfly_sum(r, lanes)
                return jnp.where(lanes == l, r, outz)

            outz = lax.fori_loop(0, L, sample_body, zero)

            fm1 = zero
            for f in range(F):
                fm1 = fm1 + fw_v[f, pl.ds(s0, L)] * vals_v[pl.ds(f * SPW + s0, L)]

            zv = outz + fm1 + btot
            out_v[pl.ds(s0, L)] = 1.0 / (1.0 + jnp.exp(-zv))
            return 0

        lax.fori_loop(0, NG, group_body, 0)
        pltpu.sync_copy(out_v, out_hbm.at[pl.ds(base, SPW)])

    return sc_kernel


def kernel(idxs, vals, shared_emb_table, fm_w_table, fm_bias,
           W1, b1, W2, b2, W3, b3):
    B, F = idxs.shape
    V, E = shared_emb_table.shape
    NW = 32
    SPW = B // NW

    weff, btot = _fold_weights(W1, W2, W3, b1, b2, b3, fm_bias)

    idx_w = idxs.reshape(NW, SPW, F).transpose(0, 2, 1)
    vals_w = vals.reshape(NW, SPW, F).transpose(0, 2, 1).reshape(NW, F * SPW)

    sc = _make_sc_kernel(B, F, E, NW)
    out_flat = sc(idx_w, vals_w, shared_emb_table, fm_w_table.reshape(-1),
                  weff.reshape(F, E), jnp.broadcast_to(btot.reshape(1), (L,)))
    return out_flat.reshape(B, 1)
